# Initial kernel scaffold; baseline (speedup 1.0000x reference)
import jax
import jax.numpy as jnp
from jax.experimental import pallas as pl


def kernel(seqs, exps, table):
    raise NotImplementedError("write your pallas kernel here")


# speedup vs baseline: 9.0776x; 9.0776x over previous
import functools

import jax
import jax.numpy as jnp
from jax import lax
from jax.experimental import pallas as pl
from jax.experimental.pallas import tpu as pltpu
from jax.experimental.pallas import tpu_sc as plsc

BATCH = 4096
SEQ_LEN = 457
EMBED_DIM = 64

NC = 2
NS = 16
NW = NC * NS

N = BATCH * SEQ_LEN
PER_W = N // NW
CHUNK = 128
NCHUNK = PER_W // CHUNK
assert PER_W * NW == N and NCHUNK * CHUNK == PER_W


def _build_gather():
    mesh = plsc.VectorSubcoreMesh(core_axis_name="c", subcore_axis_name="s")

    @functools.partial(
        pl.kernel,
        out_type=jax.ShapeDtypeStruct((N, EMBED_DIM), jnp.float32),
        mesh=mesh,
        scratch_types=[
            pltpu.VMEM((NCHUNK, CHUNK), jnp.int32),
            pltpu.VMEM((CHUNK, EMBED_DIM), jnp.float32),
            pltpu.SemaphoreType.DMA,
        ],
        compiler_params=pltpu.CompilerParams(use_tc_tiling_on_sc=False),
    )
    def gather(table_hbm, idx_hbm, out_hbm, idx_v, rows_v, gsem):
        wid = lax.axis_index("s") * NC + lax.axis_index("c")
        base = wid * PER_W
        pltpu.sync_copy(idx_hbm.at[wid], idx_v)

        @pl.loop(0, NCHUNK)
        def _(j):
            pltpu.async_copy(table_hbm.at[idx_v.at[j]], rows_v, gsem).wait()
            pltpu.sync_copy(rows_v, out_hbm.at[pl.ds(base + j * CHUNK, CHUNK)])

    return gather


_gather = _build_gather()


@jax.jit
def kernel(seqs, exps, table):
    del exps
    idx = seqs.reshape(NW, NCHUNK, CHUNK)
    out = _gather(table, idx)
    return out.reshape(BATCH, SEQ_LEN, EMBED_DIM)


# speedup vs baseline: 9.3352x; 1.0284x over previous
import functools

import jax
import jax.numpy as jnp
from jax import lax
from jax.experimental import pallas as pl
from jax.experimental.pallas import tpu as pltpu
from jax.experimental.pallas import tpu_sc as plsc

BATCH = 4096
SEQ_LEN = 457
EMBED_DIM = 64

NC = 2
NS = 16
NW = NC * NS

N = BATCH * SEQ_LEN
PER_W = N // NW
CHUNK = 128
NCHUNK = PER_W // CHUNK
assert PER_W * NW == N and NCHUNK * CHUNK == PER_W


NBUF = 4
MAIN = (NCHUNK // NBUF) * NBUF


def _build_gather():
    mesh = plsc.VectorSubcoreMesh(core_axis_name="c", subcore_axis_name="s")

    @functools.partial(
        pl.kernel,
        out_type=jax.ShapeDtypeStruct((N, EMBED_DIM), jnp.float32),
        mesh=mesh,
        scratch_types=[
            pltpu.VMEM((NCHUNK, CHUNK), jnp.int32),
            pltpu.VMEM((NBUF, CHUNK, EMBED_DIM), jnp.float32),
            pltpu.SemaphoreType.DMA((NBUF,)),
            pltpu.SemaphoreType.DMA((NBUF,)),
        ],
        compiler_params=pltpu.CompilerParams(use_tc_tiling_on_sc=False),
    )
    def gather(table_hbm, idx_hbm, out_hbm, idx_v, rows_v, gsem, wsem):
        wid = lax.axis_index("s") * NC + lax.axis_index("c")
        base = wid * PER_W
        pltpu.sync_copy(idx_hbm.at[wid], idx_v)

        def start_gather(c, b):
            pltpu.async_copy(table_hbm.at[idx_v.at[c]], rows_v.at[b], gsem.at[b])

        def wait_gather(c, b):
            pltpu.make_async_copy(
                table_hbm.at[idx_v.at[c]], rows_v.at[b], gsem.at[b]
            ).wait()

        def start_write(c, b):
            pltpu.async_copy(
                rows_v.at[b], out_hbm.at[pl.ds(base + c * CHUNK, CHUNK)], wsem.at[b]
            )

        def wait_write(c, b):
            pltpu.make_async_copy(
                rows_v.at[b], out_hbm.at[pl.ds(base + c * CHUNK, CHUNK)], wsem.at[b]
            ).wait()

        for b in range(NBUF):
            start_gather(b, b)

        @pl.loop(0, MAIN, step=NBUF)
        def _(j):
            for b in range(NBUF):
                c = j + b
                wait_gather(c, b)
                start_write(c, b)

                @pl.when(c + NBUF < NCHUNK)
                def _():
                    wait_write(c, b)
                    start_gather(c + NBUF, b)

        wait_gather(NCHUNK - 1, 0)
        start_write(NCHUNK - 1, 0)
        for c in range(MAIN - NBUF + 1, MAIN):
            wait_write(c, c % NBUF)
        wait_write(NCHUNK - 1, 0)

    return gather


_gather = _build_gather()


@jax.jit
def kernel(seqs, exps, table):
    del exps
    idx = seqs.reshape(NW, NCHUNK, CHUNK)
    out = _gather(table, idx)
    return out.reshape(BATCH, SEQ_LEN, EMBED_DIM)


# speedup vs baseline: 9.3525x; 1.0019x over previous
import functools

import jax
import jax.numpy as jnp
from jax import lax
from jax.experimental import pallas as pl
from jax.experimental.pallas import tpu as pltpu
from jax.experimental.pallas import tpu_sc as plsc

BATCH = 4096
SEQ_LEN = 457
EMBED_DIM = 64

NC = 2
NS = 16
NW = NC * NS

N = BATCH * SEQ_LEN
PER_W = N // NW
CHUNK = 128
NCHUNK = PER_W // CHUNK
assert PER_W * NW == N and NCHUNK * CHUNK == PER_W


SEQ_PER_W = BATCH // NW
SPLITS = [(0, 128), (128, 128), (256, 128), (384, 73)]


def _build_gather():
    mesh = plsc.VectorSubcoreMesh(core_axis_name="c", subcore_axis_name="s")

    @functools.partial(
        pl.kernel,
        out_type=jax.ShapeDtypeStruct((BATCH, SEQ_LEN, EMBED_DIM), jnp.float32),
        mesh=mesh,
        scratch_types=[
            pltpu.VMEM((SEQ_PER_W, SEQ_LEN), jnp.int32),
            pltpu.VMEM((2, SEQ_LEN, EMBED_DIM), jnp.float32),
            pltpu.SemaphoreType.DMA((2,)),
            pltpu.SemaphoreType.DMA((2,)),
        ],
        compiler_params=pltpu.CompilerParams(use_tc_tiling_on_sc=False),
    )
    def gather(table_hbm, idx_hbm, out_hbm, idx_v, rows_v, gsem, wsem):
        wid = lax.axis_index("s") * NC + lax.axis_index("c")
        b0 = wid * SEQ_PER_W
        pltpu.sync_copy(idx_hbm.at[pl.ds(b0, SEQ_PER_W)], idx_v)

        def start_gathers(s, p):
            for off, ln in SPLITS:
                pltpu.async_copy(
                    table_hbm.at[idx_v.at[s, pl.ds(off, ln)]],
                    rows_v.at[p, pl.ds(off, ln)],
                    gsem.at[p],
                )

        def wait_gathers(s, p):
            for off, ln in SPLITS:
                pltpu.make_async_copy(
                    table_hbm.at[idx_v.at[s, pl.ds(off, ln)]],
                    rows_v.at[p, pl.ds(off, ln)],
                    gsem.at[p],
                ).wait()

        def start_write(s, p):
            pltpu.async_copy(rows_v.at[p], out_hbm.at[b0 + s], wsem.at[p])

        def wait_write(s, p):
            pltpu.make_async_copy(
                rows_v.at[p], out_hbm.at[b0 + s], wsem.at[p]
            ).wait()

        start_gathers(0, 0)

        @pl.loop(0, SEQ_PER_W, step=2)
        def _(j):
            for p in range(2):
                s = j + p
                wait_gathers(s, p)
                start_write(s, p)

                @pl.when(s + 1 < SEQ_PER_W)
                def _():
                    @pl.when(s >= 1)
                    def _():
                        wait_write(s - 1, 1 - p)

                    start_gathers(s + 1, 1 - p)

        wait_write(SEQ_PER_W - 2, 0)
        wait_write(SEQ_PER_W - 1, 1)

    return gather


_gather = _build_gather()


@jax.jit
def kernel(seqs, exps, table):
    del exps
    return _gather(table, seqs)
